# baseline (speedup 1.0000x reference)
import jax
import jax.numpy as jnp
from jax.experimental import pallas as pl
from jax.experimental.pallas import tpu as pltpu

S, H, E = 2048, 1024, 16
TB = 256


def _moe_body(tokens_ref, gateW_ref, W_ref, b_ref, out_ref):
    e = pl.program_id(1)
    x = tokens_ref[...]
    logits = jax.lax.dot_general(
        x, gateW_ref[...], (((1,), (1,)), ((), ())),
        preferred_element_type=jnp.float32)
    w = jax.nn.softmax(logits, axis=-1)
    m0 = jnp.max(w, axis=-1, keepdims=True)
    m1 = jnp.max(jnp.where(w < m0, w, -1.0), axis=-1, keepdims=True)
    comb = jnp.where(w >= m1, w, 0.0)
    oh = (jax.lax.broadcasted_iota(jnp.int32, (1, E), 1) == e).astype(jnp.float32)
    scale = jax.lax.dot_general(comb, oh, (((1,), (1,)), ((), ())),
                                preferred_element_type=jnp.float32)
    We = W_ref[0]
    xw = jax.lax.dot_general(
        x.astype(jnp.bfloat16), We.astype(jnp.bfloat16),
        (((1,), (1,)), ((), ())), preferred_element_type=jnp.float32)
    y = jnp.maximum(xw + b_ref[0], 0.0)
    contrib = y * scale

    @pl.when(e == 0)
    def _init():
        out_ref[...] = contrib

    @pl.when(e > 0)
    def _acc():
        out_ref[...] += contrib


def kernel(tokens, gate_W, expert_W, expert_b):
    b, s, h = tokens.shape
    flat = tokens.reshape(s, h)
    out = pl.pallas_call(
        _moe_body,
        grid=(s // TB, E),
        in_specs=[
            pl.BlockSpec((TB, H), lambda t, e: (t, 0)),
            pl.BlockSpec((E, H), lambda t, e: (0, 0)),
            pl.BlockSpec((1, H, H), lambda t, e: (e, 0, 0)),
            pl.BlockSpec((1, 1, H), lambda t, e: (e, 0, 0)),
        ],
        out_specs=pl.BlockSpec((TB, H), lambda t, e: (t, 0)),
        out_shape=jax.ShapeDtypeStruct((s, h), jnp.float32),
        compiler_params=pltpu.CompilerParams(
            dimension_semantics=("parallel", "arbitrary")),
    )(flat, gate_W, expert_W, expert_b.reshape(E, 1, H))
    return out.reshape(b, s, h)


# speedup vs baseline: 2.3475x; 2.3475x over previous
import jax
import jax.numpy as jnp
from jax.experimental import pallas as pl
from jax.experimental.pallas import tpu as pltpu

S, H, E = 2048, 1024, 16


def _moe_body(tokens_ref, gateW_ref, W_ref, b_ref, out_ref, xbf_ref, comb_ref):
    e = pl.program_id(0)

    @pl.when(e == 0)
    def _gate():
        x = tokens_ref[...]
        xbf_ref[...] = x.astype(jnp.bfloat16)
        logits = jax.lax.dot_general(
            x, gateW_ref[...], (((1,), (1,)), ((), ())),
            preferred_element_type=jnp.float32)
        w = jax.nn.softmax(logits, axis=-1)
        m0 = jnp.max(w, axis=-1, keepdims=True)
        m1 = jnp.max(jnp.where(w < m0, w, -1.0), axis=-1, keepdims=True)
        comb_ref[...] = jnp.where(w >= m1, w, 0.0)
        out_ref[...] = jnp.zeros_like(out_ref)

    oh = (jax.lax.broadcasted_iota(jnp.int32, (1, E), 1) == e).astype(jnp.float32)
    scale = jax.lax.dot_general(comb_ref[...], oh, (((1,), (1,)), ((), ())),
                                preferred_element_type=jnp.float32)
    xw = jax.lax.dot_general(
        xbf_ref[...], W_ref[0].astype(jnp.bfloat16),
        (((1,), (1,)), ((), ())), preferred_element_type=jnp.float32)
    y = jnp.maximum(xw + b_ref[0], 0.0)
    out_ref[...] += y * scale


def kernel(tokens, gate_W, expert_W, expert_b):
    b, s, h = tokens.shape
    flat = tokens.reshape(s, h)
    out = pl.pallas_call(
        _moe_body,
        grid=(E,),
        in_specs=[
            pl.BlockSpec((S, H), lambda e: (0, 0)),
            pl.BlockSpec((E, H), lambda e: (0, 0)),
            pl.BlockSpec((1, H, H), lambda e: (e, 0, 0)),
            pl.BlockSpec((1, 1, H), lambda e: (e, 0, 0)),
        ],
        out_specs=pl.BlockSpec((S, H), lambda e: (0, 0)),
        out_shape=jax.ShapeDtypeStruct((s, h), jnp.float32),
        scratch_shapes=[
            pltpu.VMEM((S, H), jnp.bfloat16),
            pltpu.VMEM((S, E), jnp.float32),
        ],
        compiler_params=pltpu.CompilerParams(
            dimension_semantics=("arbitrary",)),
    )(flat, gate_W, expert_W, expert_b.reshape(E, 1, H))
    return out.reshape(b, s, h)
